# Initial kernel scaffold; baseline (speedup 1.0000x reference)
import jax
import jax.numpy as jnp
from jax.experimental import pallas as pl


def kernel(text_tokens, table):
    raise NotImplementedError("write your pallas kernel here")


# speedup vs baseline: 2.8172x; 2.8172x over previous
import functools
import math

import jax
import jax.numpy as jnp
from jax import lax
from jax.experimental import pallas as pl
from jax.experimental.pallas import tpu as pltpu
from jax.experimental.pallas import tpu_sc as plsc

_D = 128
_B = 4096
_S = 50
_TOTAL = _B * _S
_CHUNK = 128
_SCALE = math.sqrt(float(_D))

_info = plsc.get_sparse_core_info()
_NC = _info.num_cores
_NS = _info.num_subcores
_NW = _NC * _NS
_PER_W = _TOTAL // _NW
_NCH = _PER_W // _CHUNK

_mesh = plsc.VectorSubcoreMesh(core_axis_name="c", subcore_axis_name="s")


@functools.partial(
    pl.kernel,
    mesh=_mesh,
    out_type=jax.ShapeDtypeStruct((_TOTAL, _D), jnp.float32),
    scratch_types=[
        pltpu.VMEM((_NCH, _CHUNK), jnp.int32),
        pltpu.VMEM((_CHUNK, _D), jnp.float32),
        pltpu.VMEM((_CHUNK, _D), jnp.float32),
        pltpu.SemaphoreType.DMA,
        pltpu.SemaphoreType.DMA,
        pltpu.SemaphoreType.DMA,
        pltpu.SemaphoreType.DMA,
    ],
)
def _embed_sc(tok_hbm, table_hbm, out_hbm, idx_v, buf0, buf1, g0, g1, o0, o1):
    wid = lax.axis_index("s") * _NC + lax.axis_index("c")
    row0 = wid * _PER_W

    pltpu.sync_copy(tok_hbm.at[wid], idx_v)

    bufs = (buf0, buf1)
    gsems = (g0, g1)
    osems = (o0, o1)

    def gather_start(j, b):
        pltpu.make_async_copy(
            table_hbm.at[idx_v.at[j]], bufs[b], gsems[b]).start()

    def gather_wait(j, b):
        pltpu.make_async_copy(
            table_hbm.at[idx_v.at[j]], bufs[b], gsems[b]).wait()

    def out_start(j, b):
        pltpu.make_async_copy(
            bufs[b], out_hbm.at[pl.ds(row0 + j * _CHUNK, _CHUNK)],
            osems[b]).start()

    def out_wait(b):
        pltpu.make_async_copy(
            bufs[b], out_hbm.at[pl.ds(row0, _CHUNK)], osems[b]).wait()

    def scale_buf(buf):
        def row(r, carry):
            for k in range(_D // 16):
                sl = pl.ds(k * 16, 16)
                buf[r, sl] = buf[r, sl] * _SCALE
            return carry
        lax.fori_loop(0, _CHUNK, row, 0)

    gather_start(0, 0)

    def pair(t, carry):
        jo = t * 2
        for b in range(2):
            j = jo + b
            nb = 1 - b
            gather_wait(j, b)

            @pl.when(j >= 1)
            def _():
                out_wait(nb)

            @pl.when(j + 1 < _NCH)
            def _():
                gather_start(j + 1, nb)

            scale_buf(bufs[b])
            out_start(j, b)
        return carry

    lax.fori_loop(0, _NCH // 2, pair, 0)
    out_wait(1)


def kernel(text_tokens, table):
    tok = text_tokens.astype(jnp.int32).reshape(_NW, _NCH, _CHUNK)
    out = _embed_sc(tok, table)
    return out.reshape(_B, _S, _D)


# speedup vs baseline: 2.8271x; 1.0035x over previous
import functools
import math

import jax
import jax.numpy as jnp
from jax import lax
from jax.experimental import pallas as pl
from jax.experimental.pallas import tpu as pltpu
from jax.experimental.pallas import tpu_sc as plsc

_D = 128
_B = 4096
_S = 50
_TOTAL = _B * _S
_CHUNK = 128
_SCALE = math.sqrt(float(_D))

_info = plsc.get_sparse_core_info()
_NC = _info.num_cores
_NS = _info.num_subcores
_NW = _NC * _NS
_PER_W = _TOTAL // _NW
_NCH = _PER_W // _CHUNK

_mesh = plsc.VectorSubcoreMesh(core_axis_name="c", subcore_axis_name="s")


@functools.partial(
    pl.kernel,
    mesh=_mesh,
    out_type=jax.ShapeDtypeStruct((_TOTAL, _D), jnp.float32),
    scratch_types=[
        pltpu.VMEM((_NCH, _CHUNK), jnp.int32),
        pltpu.VMEM((_CHUNK, _D), jnp.float32),
        pltpu.VMEM((_CHUNK, _D), jnp.float32),
        pltpu.SemaphoreType.DMA,
        pltpu.SemaphoreType.DMA,
        pltpu.SemaphoreType.DMA,
        pltpu.SemaphoreType.DMA,
    ],
)
def _embed_sc(tok_hbm, table_hbm, out_hbm, idx_v, buf0, buf1, g0, g1, o0, o1):
    wid = lax.axis_index("s") * _NC + lax.axis_index("c")
    row0 = wid * _PER_W

    pltpu.sync_copy(tok_hbm.at[wid], idx_v)

    bufs = (buf0, buf1)
    gsems = (g0, g1)
    osems = (o0, o1)

    def gather_start(j, b):
        pltpu.make_async_copy(
            table_hbm.at[idx_v.at[j]], bufs[b], gsems[b]).start()

    def gather_wait(j, b):
        pltpu.make_async_copy(
            table_hbm.at[idx_v.at[j]], bufs[b], gsems[b]).wait()

    def out_start(j, b):
        pltpu.make_async_copy(
            bufs[b], out_hbm.at[pl.ds(row0 + j * _CHUNK, _CHUNK)],
            osems[b]).start()

    def out_wait(b):
        pltpu.make_async_copy(
            bufs[b], out_hbm.at[pl.ds(row0, _CHUNK)], osems[b]).wait()

    def scale_buf(buf):
        @plsc.parallel_loop(0, _CHUNK, step=1, unroll=8)
        def _row(r):
            for k in range(_D // 16):
                sl = pl.ds(k * 16, 16)
                buf[r, sl] = buf[r, sl] * _SCALE

    gather_start(0, 0)

    def pair(t, carry):
        jo = t * 2
        for b in range(2):
            j = jo + b
            nb = 1 - b
            gather_wait(j, b)

            @pl.when(j >= 1)
            def _():
                out_wait(nb)

            @pl.when(j + 1 < _NCH)
            def _():
                gather_start(j + 1, nb)

            scale_buf(bufs[b])
            out_start(j, b)
        return carry

    lax.fori_loop(0, _NCH // 2, pair, 0)
    out_wait(1)


def kernel(text_tokens, table):
    tok = text_tokens.astype(jnp.int32).reshape(_NW, _NCH, _CHUNK)
    out = _embed_sc(tok, table)
    return out.reshape(_B, _S, _D)


# speedup vs baseline: 3.7082x; 1.3117x over previous
import functools
import math

import jax
import jax.numpy as jnp
from jax import lax
from jax.experimental import pallas as pl
from jax.experimental.pallas import tpu as pltpu
from jax.experimental.pallas import tpu_sc as plsc

_D = 128
_B = 4096
_S = 50
_SCALE = math.sqrt(float(_D))

_info = plsc.get_sparse_core_info()
_NC = _info.num_cores
_NS = _info.num_subcores
_NW = _NC * _NS
_ENT = _B // _NW

_mesh = plsc.VectorSubcoreMesh(core_axis_name="c", subcore_axis_name="s")


@functools.partial(
    pl.kernel,
    mesh=_mesh,
    out_type=jax.ShapeDtypeStruct((_B, _S, _D), jnp.float32),
    scratch_types=[
        pltpu.VMEM((_ENT, _S), jnp.int32),
        pltpu.VMEM((_S, _D), jnp.float32),
        pltpu.VMEM((_S, _D), jnp.float32),
        pltpu.SemaphoreType.DMA,
        pltpu.SemaphoreType.DMA,
        pltpu.SemaphoreType.DMA,
        pltpu.SemaphoreType.DMA,
    ],
)
def _embed_sc(tok_hbm, table_hbm, out_hbm, idx_v, buf0, buf1, g0, g1, o0, o1):
    wid = lax.axis_index("s") * _NC + lax.axis_index("c")
    row0 = wid * _ENT

    pltpu.sync_copy(tok_hbm.at[pl.ds(row0, _ENT)], idx_v)

    bufs = (buf0, buf1)
    gsems = (g0, g1)
    osems = (o0, o1)

    def gather_start(j, b):
        pltpu.make_async_copy(
            table_hbm.at[idx_v.at[j]], bufs[b], gsems[b]).start()

    def gather_wait(j, b):
        pltpu.make_async_copy(
            table_hbm.at[idx_v.at[j]], bufs[b], gsems[b]).wait()

    def out_start(j, b):
        pltpu.make_async_copy(bufs[b], out_hbm.at[row0 + j], osems[b]).start()

    def out_wait(b):
        pltpu.make_async_copy(bufs[b], out_hbm.at[row0], osems[b]).wait()

    def scale_buf(buf):
        @plsc.parallel_loop(0, _S, step=1, unroll=10)
        def _row(r):
            for k in range(_D // 16):
                sl = pl.ds(k * 16, 16)
                buf[r, sl] = buf[r, sl] * _SCALE

    gather_start(0, 0)

    def pair(t, carry):
        jo = t * 2
        for b in range(2):
            j = jo + b
            nb = 1 - b
            gather_wait(j, b)

            @pl.when(j >= 1)
            def _():
                out_wait(nb)

            @pl.when(j + 1 < _ENT)
            def _():
                gather_start(j + 1, nb)

            scale_buf(bufs[b])
            out_start(j, b)
        return carry

    lax.fori_loop(0, _ENT // 2, pair, 0)
    out_wait(1)


def kernel(text_tokens, table):
    return _embed_sc(text_tokens.astype(jnp.int32), table)


# speedup vs baseline: 5.0842x; 1.3711x over previous
import functools
import math

import jax
import jax.numpy as jnp
from jax import lax
from jax.experimental import pallas as pl
from jax.experimental.pallas import tpu as pltpu
from jax.experimental.pallas import tpu_sc as plsc

_D = 128
_B = 4096
_S = 50
_SCALE = math.sqrt(float(_D))

_info = plsc.get_sparse_core_info()
_NC = _info.num_cores
_NS = _info.num_subcores
_NW = _NC * _NS
_ENT = _B // _NW
_G = 4
_NCH = _ENT // _G

_mesh = plsc.VectorSubcoreMesh(core_axis_name="c", subcore_axis_name="s")


@functools.partial(
    pl.kernel,
    mesh=_mesh,
    out_type=jax.ShapeDtypeStruct((_B, _S, _D), jnp.float32),
    scratch_types=[
        pltpu.VMEM((_ENT, _S), jnp.int32),
        pltpu.VMEM((_G, _S, _D), jnp.float32),
        pltpu.VMEM((_G, _S, _D), jnp.float32),
        pltpu.SemaphoreType.DMA,
        pltpu.SemaphoreType.DMA,
        pltpu.SemaphoreType.DMA,
        pltpu.SemaphoreType.DMA,
    ],
)
def _embed_sc(tok_hbm, table_hbm, out_hbm, idx_v, buf0, buf1, g0, g1, o0, o1):
    wid = lax.axis_index("s") * _NC + lax.axis_index("c")
    row0 = wid * _ENT

    pltpu.sync_copy(tok_hbm.at[pl.ds(row0, _ENT)], idx_v)

    bufs = (buf0, buf1)
    gsems = (g0, g1)
    osems = (o0, o1)

    def gather_start(j, b):
        for e in range(_G):
            pltpu.make_async_copy(
                table_hbm.at[idx_v.at[j * _G + e]], bufs[b].at[e],
                gsems[b]).start()

    def gather_wait(b):
        pltpu.make_async_copy(
            out_hbm.at[pl.ds(row0, _G)], bufs[b], gsems[b]).wait()

    def out_start(j, b):
        pltpu.make_async_copy(
            bufs[b], out_hbm.at[pl.ds(row0 + j * _G, _G)], osems[b]).start()

    def out_wait(b):
        pltpu.make_async_copy(
            bufs[b], out_hbm.at[pl.ds(row0, _G)], osems[b]).wait()

    def scale_buf(buf):
        for e in range(_G):
            @plsc.parallel_loop(0, _S, step=1, unroll=10)
            def _row(r):
                for k in range(_D // 16):
                    sl = pl.ds(k * 16, 16)
                    buf[e, r, sl] = buf[e, r, sl] * _SCALE

    gather_start(0, 0)

    def pair(t, carry):
        jo = t * 2
        for b in range(2):
            j = jo + b
            nb = 1 - b
            gather_wait(b)

            @pl.when(j >= 1)
            def _():
                out_wait(nb)

            @pl.when(j + 1 < _NCH)
            def _():
                gather_start(j + 1, nb)

            scale_buf(bufs[b])
            out_start(j, b)
        return carry

    lax.fori_loop(0, _NCH // 2, pair, 0)
    out_wait(1)


def kernel(text_tokens, table):
    return _embed_sc(text_tokens.astype(jnp.int32), table)
